# baseline (speedup 1.0000x reference)
import functools

import jax
import jax.numpy as jnp
from jax.experimental import pallas as pl
from jax.experimental.pallas import tpu as pltpu

H = 768
NH = 12
NKV = 4
HD = 64
E = 8
TOPK = 2
DFF = 512
T = 2048
THETA = 1000000.0
EPS = 1e-06
REP = NH // NKV


def _bf(x):
    return x.astype(jnp.bfloat16)


def _dot(a, b):
    return jax.lax.dot_general(
        _bf(a), _bf(b), (((1,), (0,)), ((), ())),
        preferred_element_type=jnp.float32)


def _rms(x, w):
    v = jnp.mean(jnp.square(x), axis=-1, keepdims=True)
    return x * jax.lax.rsqrt(v + EPS) * w


def _qkv_kernel(pos_ref, hs_ref, wq_ref, wk_ref, wv_ref, qn_ref, kn_ref,
                ln_ref, qo_ref, ko_ref, vo_ref):
    hs = hs_ref[...]
    h = _rms(hs, ln_ref[...])
    q = _dot(h, wq_ref[...])
    k = _dot(h, wk_ref[...])
    v = _dot(h, wv_ref[...])

    half = HD // 2
    pos = pos_ref[...].astype(jnp.float32)
    d128 = jax.lax.broadcasted_iota(jnp.int32, (1, 128), 1)
    inv128 = jnp.exp(-jnp.log(THETA) *
                     (d128 % half).astype(jnp.float32) / half)
    ang = pos * inv128
    cos128 = jnp.cos(ang)
    sin128 = jnp.sin(ang)

    def tile_lanes(x, w):
        return jnp.concatenate([x] * (w // x.shape[-1]), axis=-1)

    def norm_rope_full(x, w1, nheads):
        width = nheads * HD
        blk = (jax.lax.broadcasted_iota(jnp.int32, (width, nheads), 0) // HD
               == jax.lax.broadcasted_iota(jnp.int32, (width, nheads), 1)
               ).astype(jnp.float32)
        ms = jax.lax.dot_general(
            jnp.square(x), blk, (((1,), (0,)), ((), ())),
            preferred_element_type=jnp.float32) * (1.0 / HD)
        sf = jax.lax.rsqrt(ms + EPS)
        sfull = jax.lax.dot_general(
            sf, blk.T, (((1,), (0,)), ((), ())),
            preferred_element_type=jnp.float32)
        xn = x * sfull * tile_lanes(w1.reshape(1, HD), width)
        lane = jax.lax.broadcasted_iota(jnp.int32, (1, width), 1) % HD
        xl = pltpu.roll(xn, width - half, 1)
        xr = pltpu.roll(xn, half, 1)
        rot = jnp.where(lane < half, -xl, xr)
        cosf = tile_lanes(cos128, width)
        sinf = tile_lanes(sin128, width)
        return xn * cosf + rot * sinf

    qr = norm_rope_full(q, qn_ref[...], NH)
    kr = norm_rope_full(k, kn_ref[...], NKV)
    for hh in range(NH):
        qo_ref[hh] = qr[:, hh * HD:(hh + 1) * HD]
    for hh in range(NKV):
        ko_ref[hh] = kr[:, hh * HD:(hh + 1) * HD]
        vo_ref[hh] = v[:, hh * HD:(hh + 1) * HD]


RB = 512
NRB = T // RB


def _attn_kernel(q_ref, k_ref, v_ref, o_ref, acc_ref):
    r = pl.program_id(0)
    c = pl.program_id(1)

    @pl.when(c <= r)
    def _():
        row = r * RB + jax.lax.broadcasted_iota(jnp.int32, (RB, RB), 0)
        col = c * RB + jax.lax.broadcasted_iota(jnp.int32, (RB, RB), 1)
        mask = row >= col
        onescol = (jax.lax.broadcasted_iota(jnp.int32, (RB, HD), 1) == 0
                   ).astype(jnp.bfloat16)
        ks = [_bf(k_ref[j]) for j in range(NKV)]
        vs = [jnp.concatenate([_bf(v_ref[j]), onescol], axis=-1)
              for j in range(NKV)]
        for h in range(NH):
            j = h // REP
            qh = _bf(q_ref[h] * (HD ** -0.5))
            s = jax.lax.dot_general(qh, ks[j], (((1,), (1,)), ((), ())),
                                    preferred_element_type=jnp.float32)
            p = jnp.where(mask, jnp.exp(s), 0.0)
            pv = jax.lax.dot_general(_bf(p), vs[j], (((1,), (0,)), ((), ())),
                                     preferred_element_type=jnp.float32)
            tot = jnp.where(c == 0, pv, acc_ref[h] + pv)
            acc_ref[h] = tot

            @pl.when(c == r)
            def _():
                o_ref[h] = tot[:, :HD] / tot[:, HD:HD + 1]


def _post_kernel(ao_ref, wo_ref, hs_ref, ln_ref, wg_ref,
                 h2_ref, h3_ref, g_ref):
    ao = jnp.concatenate([ao_ref[hh] for hh in range(NH)], axis=-1)
    attn_out = _dot(ao, wo_ref[...])
    h2 = attn_out + hs_ref[...]
    h3 = _rms(h2, ln_ref[...])
    h2_ref[...] = h2
    h3_ref[...] = h3

    logits = jax.lax.dot_general(
        h3, wg_ref[...], (((1,), (0,)), ((), ())),
        preferred_element_type=jnp.float32)
    scores = jax.nn.sigmoid(logits)
    idx = jax.lax.broadcasted_iota(jnp.int32, (T, E), 1)
    m1 = jnp.max(scores, axis=-1, keepdims=True)
    i1 = jnp.min(jnp.where(scores == m1, idx, E), axis=-1, keepdims=True)
    s2 = jnp.where(idx == i1, -jnp.inf, scores)
    m2 = jnp.max(s2, axis=-1, keepdims=True)
    i2 = jnp.min(jnp.where(s2 == m2, idx, E), axis=-1, keepdims=True)
    denom = m1 + m2 + 1e-20
    g = jnp.where(idx == i1, m1 / denom, 0.0) + \
        jnp.where(idx == i2, m2 / denom, 0.0)
    g_ref[...] = g


def _silu(x):
    return x * jax.nn.sigmoid(x)


def _moe_kernel(h3_ref, g_ref, h2_ref, eg_ref, eu_ref, ed_ref,
                sg_ref, su_ref, sd_ref, o_ref):
    e = pl.program_id(0)
    x = _bf(h3_ref[...])

    def mlp(g_w, u_w, d_w):
        gate = _bf(jax.lax.dot_general(x, _bf(g_w), (((1,), (0,)), ((), ())),
                                       preferred_element_type=jnp.float32))
        up = _bf(jax.lax.dot_general(x, _bf(u_w), (((1,), (0,)), ((), ())),
                                     preferred_element_type=jnp.float32))
        return jax.lax.dot_general(_silu(gate) * up, _bf(d_w),
                                   (((1,), (0,)), ((), ())),
                                   preferred_element_type=jnp.float32)

    @pl.when(e == 0)
    def _():
        o_ref[...] = h2_ref[...] + mlp(sg_ref[...], su_ref[...], sd_ref[...])

    y = mlp(eg_ref[0], eu_ref[0], ed_ref[0])
    lane = jax.lax.broadcasted_iota(jnp.int32, (T, E), 1)
    w = jnp.sum(jnp.where(lane == e, g_ref[...], 0.0), axis=-1, keepdims=True)
    o_ref[...] += w * y


@functools.partial(jax.jit, static_argnames=())
def kernel(positions, hidden_states, Wq, Wk, Wv, Wo, q_norm_w, k_norm_w,
           in_ln_w, post_ln_w, Wg, We_gate, We_up, We_down, Ws_gate, Ws_up,
           Ws_down):
    pos2d = positions.reshape(T, 1)

    qkv = pl.pallas_call(
        _qkv_kernel,
        out_shape=(
            jax.ShapeDtypeStruct((NH, T, HD), jnp.float32),
            jax.ShapeDtypeStruct((NKV, T, HD), jnp.float32),
            jax.ShapeDtypeStruct((NKV, T, HD), jnp.float32),
        ),
    )
    q, k, v = qkv(pos2d, hidden_states, Wq, Wk, Wv, q_norm_w, k_norm_w,
                  in_ln_w)

    kv_idx = lambda r, c: (0, jnp.minimum(c, r), 0)
    ao = pl.pallas_call(
        _attn_kernel,
        grid=(NRB, NRB),
        in_specs=[
            pl.BlockSpec((NH, RB, HD), lambda r, c: (0, r, 0)),
            pl.BlockSpec((NKV, RB, HD), kv_idx),
            pl.BlockSpec((NKV, RB, HD), kv_idx),
        ],
        out_specs=pl.BlockSpec((NH, RB, HD), lambda r, c: (0, r, 0)),
        out_shape=jax.ShapeDtypeStruct((NH, T, HD), jnp.float32),
        scratch_shapes=[pltpu.VMEM((NH, RB, 2 * HD), jnp.float32)],
    )(q, k, v)

    h2, h3, g = pl.pallas_call(
        _post_kernel,
        out_shape=(
            jax.ShapeDtypeStruct((T, H), jnp.float32),
            jax.ShapeDtypeStruct((T, H), jnp.float32),
            jax.ShapeDtypeStruct((T, E), jnp.float32),
        ),
    )(ao, Wo, hidden_states, post_ln_w, Wg)

    out = pl.pallas_call(
        _moe_kernel,
        grid=(E,),
        in_specs=[
            pl.BlockSpec((T, H), lambda e: (0, 0)),
            pl.BlockSpec((T, E), lambda e: (0, 0)),
            pl.BlockSpec((T, H), lambda e: (0, 0)),
            pl.BlockSpec((1, H, DFF), lambda e: (e, 0, 0)),
            pl.BlockSpec((1, H, DFF), lambda e: (e, 0, 0)),
            pl.BlockSpec((1, DFF, H), lambda e: (e, 0, 0)),
            pl.BlockSpec((H, DFF), lambda e: (0, 0)),
            pl.BlockSpec((H, DFF), lambda e: (0, 0)),
            pl.BlockSpec((DFF, H), lambda e: (0, 0)),
        ],
        out_specs=pl.BlockSpec((T, H), lambda e: (0, 0)),
        out_shape=jax.ShapeDtypeStruct((T, H), jnp.float32),
    )(h3, g, h2, We_gate, We_up, We_down, Ws_gate, Ws_up, Ws_down)

    return out


# speedup vs baseline: 1.1178x; 1.1178x over previous
import functools

import jax
import jax.numpy as jnp
from jax.experimental import pallas as pl
from jax.experimental.pallas import tpu as pltpu

H = 768
NH = 12
NKV = 4
HD = 64
E = 8
TOPK = 2
DFF = 512
T = 2048
THETA = 1000000.0
EPS = 1e-06
REP = NH // NKV


def _bf(x):
    return x.astype(jnp.bfloat16)


def _dot(a, b):
    return jax.lax.dot_general(
        _bf(a), _bf(b), (((1,), (0,)), ((), ())),
        preferred_element_type=jnp.float32)


def _rms(x, w):
    v = jnp.mean(jnp.square(x), axis=-1, keepdims=True)
    return x * jax.lax.rsqrt(v + EPS) * w


def _qkv_kernel(pos_ref, hs_ref, wq_ref, wk_ref, wv_ref, qn_ref, kn_ref,
                ln_ref, qo_ref, ko_ref, vo_ref):
    hs = hs_ref[...]
    h = _rms(hs, ln_ref[...])
    q = _dot(h, wq_ref[...])
    k = _dot(h, wk_ref[...])
    v = _dot(h, wv_ref[...])

    half = HD // 2
    pos = pos_ref[...].astype(jnp.float32)
    d128 = jax.lax.broadcasted_iota(jnp.int32, (1, 128), 1)
    inv128 = jnp.exp(-jnp.log(THETA) *
                     (d128 % half).astype(jnp.float32) / half)
    ang = pos * inv128
    cos128 = jnp.cos(ang)
    sin128 = jnp.sin(ang)

    def tile_lanes(x, w):
        return jnp.concatenate([x] * (w // x.shape[-1]), axis=-1)

    def norm_rope_full(x, w1, nheads):
        width = nheads * HD
        blk = (jax.lax.broadcasted_iota(jnp.int32, (width, nheads), 0) // HD
               == jax.lax.broadcasted_iota(jnp.int32, (width, nheads), 1)
               ).astype(jnp.float32)
        ms = jax.lax.dot_general(
            jnp.square(x), blk, (((1,), (0,)), ((), ())),
            preferred_element_type=jnp.float32) * (1.0 / HD)
        sf = jax.lax.rsqrt(ms + EPS)
        sfull = jax.lax.dot_general(
            sf, blk.T, (((1,), (0,)), ((), ())),
            preferred_element_type=jnp.float32)
        xn = x * sfull * tile_lanes(w1.reshape(1, HD), width)
        lane = jax.lax.broadcasted_iota(jnp.int32, (1, width), 1) % HD
        xl = pltpu.roll(xn, width - half, 1)
        xr = pltpu.roll(xn, half, 1)
        rot = jnp.where(lane < half, -xl, xr)
        cosf = tile_lanes(cos128, width)
        sinf = tile_lanes(sin128, width)
        return xn * cosf + rot * sinf

    qr = _bf(norm_rope_full(q, qn_ref[...], NH) * (HD ** -0.5))
    kr = _bf(norm_rope_full(k, kn_ref[...], NKV))
    vb = _bf(v)
    for hh in range(NH):
        qo_ref[hh] = qr[:, hh * HD:(hh + 1) * HD]
    for hh in range(NKV):
        ko_ref[hh] = kr[:, hh * HD:(hh + 1) * HD]
        vo_ref[hh] = vb[:, hh * HD:(hh + 1) * HD]


RB = 512
NRB = T // RB


def _attn_kernel(q_ref, k_ref, v_ref, o_ref):
    r = pl.program_id(1)
    q = q_ref[0]

    def block(c, masked):
        k = k_ref[0, pl.ds(c * RB, RB), :]
        v = v_ref[0, pl.ds(c * RB, RB), :]
        s = jax.lax.dot_general(q, k, (((1,), (1,)), ((), ())),
                                preferred_element_type=jnp.float32)
        p = jnp.exp(s)
        if masked:
            row = jax.lax.broadcasted_iota(jnp.int32, (RB, RB), 0)
            col = jax.lax.broadcasted_iota(jnp.int32, (RB, RB), 1)
            p = jnp.where(row >= col, p, 0.0)
        pv = jax.lax.dot_general(_bf(p), v, (((1,), (0,)), ((), ())),
                                 preferred_element_type=jnp.float32)
        return pv, jnp.sum(p, axis=-1, keepdims=True)

    def body(c, carry):
        acc, denom = carry
        pv, ps = block(c, masked=False)
        return acc + pv, denom + ps

    acc, denom = block(r, masked=True)
    acc, denom = jax.lax.fori_loop(0, r, body, (acc, denom))
    o_ref[0] = _bf(acc / denom)


def _post_kernel(ao_ref, wo_ref, hs_ref, ln_ref, wg_ref,
                 h2_ref, h3_ref, g_ref):
    ao = jnp.concatenate([ao_ref[hh] for hh in range(NH)], axis=-1)
    attn_out = _dot(ao, wo_ref[...])
    h2 = attn_out + hs_ref[...]
    h3 = _rms(h2, ln_ref[...])
    h2_ref[...] = h2
    h3_ref[...] = h3

    logits = jax.lax.dot_general(
        h3, wg_ref[...], (((1,), (0,)), ((), ())),
        preferred_element_type=jnp.float32)
    scores = jax.nn.sigmoid(logits)
    idx = jax.lax.broadcasted_iota(jnp.int32, (T, E), 1)
    m1 = jnp.max(scores, axis=-1, keepdims=True)
    i1 = jnp.min(jnp.where(scores == m1, idx, E), axis=-1, keepdims=True)
    s2 = jnp.where(idx == i1, -jnp.inf, scores)
    m2 = jnp.max(s2, axis=-1, keepdims=True)
    i2 = jnp.min(jnp.where(s2 == m2, idx, E), axis=-1, keepdims=True)
    denom = m1 + m2 + 1e-20
    g = jnp.where(idx == i1, m1 / denom, 0.0) + \
        jnp.where(idx == i2, m2 / denom, 0.0)
    g_ref[...] = g


def _silu(x):
    return x * jax.nn.sigmoid(x)


def _moe_kernel(h3_ref, g_ref, h2_ref, eg_ref, eu_ref, ed_ref,
                sg_ref, su_ref, sd_ref, o_ref):
    e = pl.program_id(0)
    x = _bf(h3_ref[...])

    def mlp(g_w, u_w, d_w):
        gate = _bf(jax.lax.dot_general(x, _bf(g_w), (((1,), (0,)), ((), ())),
                                       preferred_element_type=jnp.float32))
        up = _bf(jax.lax.dot_general(x, _bf(u_w), (((1,), (0,)), ((), ())),
                                     preferred_element_type=jnp.float32))
        return jax.lax.dot_general(_silu(gate) * up, _bf(d_w),
                                   (((1,), (0,)), ((), ())),
                                   preferred_element_type=jnp.float32)

    @pl.when(e == 0)
    def _():
        o_ref[...] = h2_ref[...] + mlp(sg_ref[...], su_ref[...], sd_ref[...])

    y = mlp(eg_ref[0], eu_ref[0], ed_ref[0])
    lane = jax.lax.broadcasted_iota(jnp.int32, (T, E), 1)
    w = jnp.sum(jnp.where(lane == e, g_ref[...], 0.0), axis=-1, keepdims=True)
    o_ref[...] += w * y


@functools.partial(jax.jit, static_argnames=())
def kernel(positions, hidden_states, Wq, Wk, Wv, Wo, q_norm_w, k_norm_w,
           in_ln_w, post_ln_w, Wg, We_gate, We_up, We_down, Ws_gate, Ws_up,
           Ws_down):
    pos2d = positions.reshape(T, 1)

    qkv = pl.pallas_call(
        _qkv_kernel,
        out_shape=(
            jax.ShapeDtypeStruct((NH, T, HD), jnp.bfloat16),
            jax.ShapeDtypeStruct((NKV, T, HD), jnp.bfloat16),
            jax.ShapeDtypeStruct((NKV, T, HD), jnp.bfloat16),
        ),
    )
    q, k, v = qkv(pos2d, hidden_states, Wq, Wk, Wv, q_norm_w, k_norm_w,
                  in_ln_w)

    ao = pl.pallas_call(
        _attn_kernel,
        grid=(NH, NRB),
        in_specs=[
            pl.BlockSpec((1, RB, HD), lambda h, r: (h, r, 0)),
            pl.BlockSpec((1, T, HD), lambda h, r: (h // REP, 0, 0)),
            pl.BlockSpec((1, T, HD), lambda h, r: (h // REP, 0, 0)),
        ],
        out_specs=pl.BlockSpec((1, RB, HD), lambda h, r: (h, r, 0)),
        out_shape=jax.ShapeDtypeStruct((NH, T, HD), jnp.bfloat16),
    )(q, k, v)

    h2, h3, g = pl.pallas_call(
        _post_kernel,
        out_shape=(
            jax.ShapeDtypeStruct((T, H), jnp.float32),
            jax.ShapeDtypeStruct((T, H), jnp.float32),
            jax.ShapeDtypeStruct((T, E), jnp.float32),
        ),
    )(ao, Wo, hidden_states, post_ln_w, Wg)

    out = pl.pallas_call(
        _moe_kernel,
        grid=(E,),
        in_specs=[
            pl.BlockSpec((T, H), lambda e: (0, 0)),
            pl.BlockSpec((T, E), lambda e: (0, 0)),
            pl.BlockSpec((T, H), lambda e: (0, 0)),
            pl.BlockSpec((1, H, DFF), lambda e: (e, 0, 0)),
            pl.BlockSpec((1, H, DFF), lambda e: (e, 0, 0)),
            pl.BlockSpec((1, DFF, H), lambda e: (e, 0, 0)),
            pl.BlockSpec((H, DFF), lambda e: (0, 0)),
            pl.BlockSpec((H, DFF), lambda e: (0, 0)),
            pl.BlockSpec((DFF, H), lambda e: (0, 0)),
        ],
        out_specs=pl.BlockSpec((T, H), lambda e: (0, 0)),
        out_shape=jax.ShapeDtypeStruct((T, H), jnp.float32),
    )(h3, g, h2, We_gate, We_up, We_down, Ws_gate, Ws_up, Ws_down)

    return out


# speedup vs baseline: 1.1954x; 1.0695x over previous
import functools

import jax
import jax.numpy as jnp
from jax.experimental import pallas as pl
from jax.experimental.pallas import tpu as pltpu

H = 768
NH = 12
NKV = 4
HD = 64
E = 8
TOPK = 2
DFF = 512
T = 2048
THETA = 1000000.0
EPS = 1e-06
REP = NH // NKV


def _bf(x):
    return x.astype(jnp.bfloat16)


def _dot(a, b):
    return jax.lax.dot_general(
        _bf(a), _bf(b), (((1,), (0,)), ((), ())),
        preferred_element_type=jnp.float32)


def _rms(x, w):
    v = jnp.mean(jnp.square(x), axis=-1, keepdims=True)
    return x * jax.lax.rsqrt(v + EPS) * w


def _qkv_kernel(pos_ref, hs_ref, wq_ref, wk_ref, wv_ref, qn_ref, kn_ref,
                ln_ref, qo_ref, ko_ref, vo_ref):
    hs = hs_ref[...]
    h = _rms(hs, ln_ref[...])
    q = _dot(h, wq_ref[...])
    k = _dot(h, wk_ref[...])
    v = _dot(h, wv_ref[...])

    half = HD // 2
    pos = pos_ref[...].astype(jnp.float32)
    d128 = jax.lax.broadcasted_iota(jnp.int32, (1, 128), 1)
    inv128 = jnp.exp(-jnp.log(THETA) *
                     (d128 % half).astype(jnp.float32) / half)
    ang = pos * inv128
    cos128 = jnp.cos(ang)
    sin128 = jnp.sin(ang)

    def tile_lanes(x, w):
        return jnp.concatenate([x] * (w // x.shape[-1]), axis=-1)

    def norm_rope_full(x, w1, nheads):
        width = nheads * HD
        blk = (jax.lax.broadcasted_iota(jnp.int32, (width, nheads), 0) // HD
               == jax.lax.broadcasted_iota(jnp.int32, (width, nheads), 1)
               ).astype(jnp.float32)
        ms = jax.lax.dot_general(
            jnp.square(x), blk, (((1,), (0,)), ((), ())),
            preferred_element_type=jnp.float32) * (1.0 / HD)
        sf = jax.lax.rsqrt(ms + EPS)
        sfull = jax.lax.dot_general(
            sf, blk.T, (((1,), (0,)), ((), ())),
            preferred_element_type=jnp.float32)
        xn = x * sfull * tile_lanes(w1.reshape(1, HD), width)
        lane = jax.lax.broadcasted_iota(jnp.int32, (1, width), 1) % HD
        xl = pltpu.roll(xn, width - half, 1)
        xr = pltpu.roll(xn, half, 1)
        rot = jnp.where(lane < half, -xl, xr)
        cosf = tile_lanes(cos128, width)
        sinf = tile_lanes(sin128, width)
        return xn * cosf + rot * sinf

    qr = _bf(norm_rope_full(q, qn_ref[...], NH) * (HD ** -0.5))
    kr = _bf(norm_rope_full(k, kn_ref[...], NKV))
    vb = _bf(v)
    for hh in range(NH):
        qo_ref[hh] = qr[:, hh * HD:(hh + 1) * HD]
    for hh in range(NKV):
        ko_ref[hh] = kr[:, hh * HD:(hh + 1) * HD]
        vo_ref[hh] = vb[:, hh * HD:(hh + 1) * HD]


RB = 512
NRB = T // RB


def _attn_kernel(q_ref, k_ref, v_ref, o_ref):
    r = pl.program_id(1)
    q = q_ref[0]

    def block(c, masked):
        k = k_ref[0, pl.ds(c * RB, RB), :]
        v = v_ref[0, pl.ds(c * RB, RB), :]
        s = jax.lax.dot_general(q, k, (((1,), (1,)), ((), ())),
                                preferred_element_type=jnp.float32)
        p = jnp.exp(s)
        if masked:
            row = jax.lax.broadcasted_iota(jnp.int32, (RB, RB), 0)
            col = jax.lax.broadcasted_iota(jnp.int32, (RB, RB), 1)
            p = jnp.where(row >= col, p, 0.0)
        pv = jax.lax.dot_general(_bf(p), v, (((1,), (0,)), ((), ())),
                                 preferred_element_type=jnp.float32)
        return pv, jnp.sum(p, axis=-1, keepdims=True)

    def body(c, carry):
        acc, denom = carry
        pv, ps = block(c, masked=False)
        return acc + pv, denom + ps

    acc, denom = block(r, masked=True)
    acc, denom = jax.lax.fori_loop(0, r, body, (acc, denom))
    o_ref[0] = _bf(acc / denom)


def _post_kernel(ao_ref, wo_ref, hs_ref, ln_ref, wg_ref,
                 h2_ref, h3_ref, g_ref):
    ao = jnp.concatenate([ao_ref[hh] for hh in range(NH)], axis=-1)
    attn_out = _dot(ao, wo_ref[...])
    h2 = attn_out + hs_ref[...]
    h3 = _rms(h2, ln_ref[...])
    h2_ref[...] = h2
    h3_ref[...] = h3

    logits = jax.lax.dot_general(
        h3, wg_ref[...], (((1,), (0,)), ((), ())),
        preferred_element_type=jnp.float32)
    scores = jax.nn.sigmoid(logits)
    idx = jax.lax.broadcasted_iota(jnp.int32, (T, E), 1)
    m1 = jnp.max(scores, axis=-1, keepdims=True)
    i1 = jnp.min(jnp.where(scores == m1, idx, E), axis=-1, keepdims=True)
    s2 = jnp.where(idx == i1, -jnp.inf, scores)
    m2 = jnp.max(s2, axis=-1, keepdims=True)
    i2 = jnp.min(jnp.where(s2 == m2, idx, E), axis=-1, keepdims=True)
    denom = m1 + m2 + 1e-20
    g = jnp.where(idx == i1, m1 / denom, 0.0) + \
        jnp.where(idx == i2, m2 / denom, 0.0)
    g_ref[...] = g


def _silu(x):
    return x * jax.nn.sigmoid(x)


def _moe_kernel(ao_ref, wo_ref, hs_ref, ln_ref, wg_ref,
                eg_ref, eu_ref, ed_ref, sg_ref, su_ref, sd_ref,
                o_ref, xb_ref, g_ref):
    e = pl.program_id(0)

    @pl.when(e == 0)
    def _():
        ao = jnp.concatenate([ao_ref[hh] for hh in range(NH)], axis=-1)
        attn_out = _dot(ao, wo_ref[...])
        h2 = attn_out + hs_ref[...]
        h3 = _rms(h2, ln_ref[...])
        xb_ref[...] = _bf(h3)

        logits = jax.lax.dot_general(
            h3, wg_ref[...], (((1,), (0,)), ((), ())),
            preferred_element_type=jnp.float32)
        scores = jax.nn.sigmoid(logits)
        idx = jax.lax.broadcasted_iota(jnp.int32, (T, E), 1)
        m1 = jnp.max(scores, axis=-1, keepdims=True)
        i1 = jnp.min(jnp.where(scores == m1, idx, E), axis=-1, keepdims=True)
        s2 = jnp.where(idx == i1, -jnp.inf, scores)
        m2 = jnp.max(s2, axis=-1, keepdims=True)
        i2 = jnp.min(jnp.where(s2 == m2, idx, E), axis=-1, keepdims=True)
        denom = m1 + m2 + 1e-20
        g_ref[...] = jnp.where(idx == i1, m1 / denom, 0.0) + \
            jnp.where(idx == i2, m2 / denom, 0.0)

        x0 = xb_ref[...]
        gate = _bf(jax.lax.dot_general(
            x0, _bf(sg_ref[...]), (((1,), (0,)), ((), ())),
            preferred_element_type=jnp.float32))
        up = _bf(jax.lax.dot_general(
            x0, _bf(su_ref[...]), (((1,), (0,)), ((), ())),
            preferred_element_type=jnp.float32))
        o_ref[...] = h2 + jax.lax.dot_general(
            _silu(gate) * up, _bf(sd_ref[...]), (((1,), (0,)), ((), ())),
            preferred_element_type=jnp.float32)

    x = xb_ref[...]
    gate = _bf(jax.lax.dot_general(x, _bf(eg_ref[0]), (((1,), (0,)), ((), ())),
                                   preferred_element_type=jnp.float32))
    up = _bf(jax.lax.dot_general(x, _bf(eu_ref[0]), (((1,), (0,)), ((), ())),
                                 preferred_element_type=jnp.float32))
    y = jax.lax.dot_general(_silu(gate) * up, _bf(ed_ref[0]),
                            (((1,), (0,)), ((), ())),
                            preferred_element_type=jnp.float32)
    lane = jax.lax.broadcasted_iota(jnp.int32, (T, E), 1)
    w = jnp.sum(jnp.where(lane == e, g_ref[...], 0.0), axis=-1, keepdims=True)
    o_ref[...] += w * y


@functools.partial(jax.jit, static_argnames=())
def kernel(positions, hidden_states, Wq, Wk, Wv, Wo, q_norm_w, k_norm_w,
           in_ln_w, post_ln_w, Wg, We_gate, We_up, We_down, Ws_gate, Ws_up,
           Ws_down):
    pos2d = positions.reshape(T, 1)

    qkv = pl.pallas_call(
        _qkv_kernel,
        out_shape=(
            jax.ShapeDtypeStruct((NH, T, HD), jnp.bfloat16),
            jax.ShapeDtypeStruct((NKV, T, HD), jnp.bfloat16),
            jax.ShapeDtypeStruct((NKV, T, HD), jnp.bfloat16),
        ),
    )
    q, k, v = qkv(pos2d, hidden_states, Wq, Wk, Wv, q_norm_w, k_norm_w,
                  in_ln_w)

    ao = pl.pallas_call(
        _attn_kernel,
        grid=(NH, NRB),
        in_specs=[
            pl.BlockSpec((1, RB, HD), lambda h, r: (h, r, 0)),
            pl.BlockSpec((1, T, HD), lambda h, r: (h // REP, 0, 0)),
            pl.BlockSpec((1, T, HD), lambda h, r: (h // REP, 0, 0)),
        ],
        out_specs=pl.BlockSpec((1, RB, HD), lambda h, r: (h, r, 0)),
        out_shape=jax.ShapeDtypeStruct((NH, T, HD), jnp.bfloat16),
    )(q, k, v)

    out = pl.pallas_call(
        _moe_kernel,
        grid=(E,),
        in_specs=[
            pl.BlockSpec((NH, T, HD), lambda e: (0, 0, 0)),
            pl.BlockSpec((NH * HD, H), lambda e: (0, 0)),
            pl.BlockSpec((T, H), lambda e: (0, 0)),
            pl.BlockSpec((H,), lambda e: (0,)),
            pl.BlockSpec((H, E), lambda e: (0, 0)),
            pl.BlockSpec((1, H, DFF), lambda e: (e, 0, 0)),
            pl.BlockSpec((1, H, DFF), lambda e: (e, 0, 0)),
            pl.BlockSpec((1, DFF, H), lambda e: (e, 0, 0)),
            pl.BlockSpec((H, DFF), lambda e: (0, 0)),
            pl.BlockSpec((H, DFF), lambda e: (0, 0)),
            pl.BlockSpec((DFF, H), lambda e: (0, 0)),
        ],
        out_specs=pl.BlockSpec((T, H), lambda e: (0, 0)),
        out_shape=jax.ShapeDtypeStruct((T, H), jnp.float32),
        scratch_shapes=[pltpu.VMEM((T, H), jnp.bfloat16),
                        pltpu.VMEM((T, E), jnp.float32)],
    )(ao, Wo, hidden_states, post_ln_w, Wg,
      We_gate, We_up, We_down, Ws_gate, Ws_up, Ws_down)

    return out


# speedup vs baseline: 1.2399x; 1.0372x over previous
import functools

import jax
import jax.numpy as jnp
from jax.experimental import pallas as pl
from jax.experimental.pallas import tpu as pltpu

H = 768
NH = 12
NKV = 4
HD = 64
E = 8
TOPK = 2
DFF = 512
T = 2048
THETA = 1000000.0
EPS = 1e-06
REP = NH // NKV


def _bf(x):
    return x.astype(jnp.bfloat16)


def _dot(a, b):
    return jax.lax.dot_general(
        _bf(a), _bf(b), (((1,), (0,)), ((), ())),
        preferred_element_type=jnp.float32)


def _rms(x, w):
    v = jnp.mean(jnp.square(x), axis=-1, keepdims=True)
    return x * jax.lax.rsqrt(v + EPS) * w


def _qkv_body(pos_ref, hs_ref, wq_ref, wk_ref, wv_ref, qn_ref, kn_ref,
                ln_ref, qo_ref, ko_ref, vo_ref):
    hs = hs_ref[...]
    h = _rms(hs, ln_ref[...])
    q = _dot(h, wq_ref[...])
    k = _dot(h, wk_ref[...])
    v = _dot(h, wv_ref[...])

    half = HD // 2
    pos = pos_ref[...].astype(jnp.float32)
    d128 = jax.lax.broadcasted_iota(jnp.int32, (1, 128), 1)
    inv128 = jnp.exp(-jnp.log(THETA) *
                     (d128 % half).astype(jnp.float32) / half)
    ang = pos * inv128
    cos128 = jnp.cos(ang)
    sin128 = jnp.sin(ang)

    def tile_lanes(x, w):
        return jnp.concatenate([x] * (w // x.shape[-1]), axis=-1)

    def norm_rope_full(x, w1, nheads):
        width = nheads * HD
        blk = (jax.lax.broadcasted_iota(jnp.int32, (width, nheads), 0) // HD
               == jax.lax.broadcasted_iota(jnp.int32, (width, nheads), 1)
               ).astype(jnp.float32)
        ms = jax.lax.dot_general(
            jnp.square(x), blk, (((1,), (0,)), ((), ())),
            preferred_element_type=jnp.float32) * (1.0 / HD)
        sf = jax.lax.rsqrt(ms + EPS)
        sfull = jax.lax.dot_general(
            sf, blk.T, (((1,), (0,)), ((), ())),
            preferred_element_type=jnp.float32)
        xn = x * sfull * tile_lanes(w1.reshape(1, HD), width)
        lane = jax.lax.broadcasted_iota(jnp.int32, (1, width), 1) % HD
        xl = pltpu.roll(xn, width - half, 1)
        xr = pltpu.roll(xn, half, 1)
        rot = jnp.where(lane < half, -xl, xr)
        cosf = tile_lanes(cos128, width)
        sinf = tile_lanes(sin128, width)
        return xn * cosf + rot * sinf

    qr = _bf(norm_rope_full(q, qn_ref[...], NH) * (HD ** -0.5))
    kr = _bf(norm_rope_full(k, kn_ref[...], NKV))
    vb = _bf(v)
    for hh in range(NH):
        qo_ref[hh] = qr[:, hh * HD:(hh + 1) * HD]
    for hh in range(NKV):
        ko_ref[hh] = kr[:, hh * HD:(hh + 1) * HD]
        vo_ref[hh] = vb[:, hh * HD:(hh + 1) * HD]


RB = 512
NRB = T // RB


def _attn_kernel(pos_ref, hs_ref, wq_ref, wk_ref, wv_ref, qn_ref, kn_ref,
                 ln_ref, o_ref, qsc_ref, ksc_ref, vsc_ref):
    h = pl.program_id(0)
    r = pl.program_id(1)

    @pl.when((h == 0) & (r == 0))
    def _():
        _qkv_body(pos_ref, hs_ref, wq_ref, wk_ref, wv_ref, qn_ref, kn_ref,
                  ln_ref, qsc_ref, ksc_ref, vsc_ref)

    j = h // REP
    q = qsc_ref[h, pl.ds(r * RB, RB), :]

    def block(c, masked):
        k = ksc_ref[j, pl.ds(c * RB, RB), :]
        v = vsc_ref[j, pl.ds(c * RB, RB), :]
        s = jax.lax.dot_general(q, k, (((1,), (1,)), ((), ())),
                                preferred_element_type=jnp.float32)
        p = jnp.exp(s)
        if masked:
            row = jax.lax.broadcasted_iota(jnp.int32, (RB, RB), 0)
            col = jax.lax.broadcasted_iota(jnp.int32, (RB, RB), 1)
            p = jnp.where(row >= col, p, 0.0)
        pv = jax.lax.dot_general(_bf(p), v, (((1,), (0,)), ((), ())),
                                 preferred_element_type=jnp.float32)
        return pv, jnp.sum(p, axis=-1, keepdims=True)

    def body(c, carry):
        acc, denom = carry
        pv, ps = block(c, masked=False)
        return acc + pv, denom + ps

    acc, denom = block(r, masked=True)
    acc, denom = jax.lax.fori_loop(0, r, body, (acc, denom))
    o_ref[0] = _bf(acc / denom)


def _silu(x):
    return x * jax.nn.sigmoid(x)


def _moe_kernel(ao_ref, wo_ref, hs_ref, ln_ref, wg_ref,
                eg_ref, eu_ref, ed_ref, sg_ref, su_ref, sd_ref,
                o_ref, xb_ref, g_ref):
    e = pl.program_id(0)

    @pl.when(e == 0)
    def _():
        ao = jnp.concatenate([ao_ref[hh] for hh in range(NH)], axis=-1)
        attn_out = _dot(ao, wo_ref[...])
        h2 = attn_out + hs_ref[...]
        h3 = _rms(h2, ln_ref[...])
        xb_ref[...] = _bf(h3)

        logits = jax.lax.dot_general(
            h3, wg_ref[...], (((1,), (0,)), ((), ())),
            preferred_element_type=jnp.float32)
        scores = jax.nn.sigmoid(logits)
        idx = jax.lax.broadcasted_iota(jnp.int32, (T, E), 1)
        m1 = jnp.max(scores, axis=-1, keepdims=True)
        i1 = jnp.min(jnp.where(scores == m1, idx, E), axis=-1, keepdims=True)
        s2 = jnp.where(idx == i1, -jnp.inf, scores)
        m2 = jnp.max(s2, axis=-1, keepdims=True)
        i2 = jnp.min(jnp.where(s2 == m2, idx, E), axis=-1, keepdims=True)
        denom = m1 + m2 + 1e-20
        g_ref[...] = jnp.where(idx == i1, m1 / denom, 0.0) + \
            jnp.where(idx == i2, m2 / denom, 0.0)

        x0 = xb_ref[...]
        gate = _bf(jax.lax.dot_general(
            x0, _bf(sg_ref[...]), (((1,), (0,)), ((), ())),
            preferred_element_type=jnp.float32))
        up = _bf(jax.lax.dot_general(
            x0, _bf(su_ref[...]), (((1,), (0,)), ((), ())),
            preferred_element_type=jnp.float32))
        o_ref[...] = h2 + jax.lax.dot_general(
            _silu(gate) * up, _bf(sd_ref[...]), (((1,), (0,)), ((), ())),
            preferred_element_type=jnp.float32)

    x = xb_ref[...]
    gate = _bf(jax.lax.dot_general(x, _bf(eg_ref[0]), (((1,), (0,)), ((), ())),
                                   preferred_element_type=jnp.float32))
    up = _bf(jax.lax.dot_general(x, _bf(eu_ref[0]), (((1,), (0,)), ((), ())),
                                 preferred_element_type=jnp.float32))
    y = jax.lax.dot_general(_silu(gate) * up, _bf(ed_ref[0]),
                            (((1,), (0,)), ((), ())),
                            preferred_element_type=jnp.float32)
    lane = jax.lax.broadcasted_iota(jnp.int32, (T, E), 1)
    w = jnp.sum(jnp.where(lane == e, g_ref[...], 0.0), axis=-1, keepdims=True)
    o_ref[...] += w * y


@functools.partial(jax.jit, static_argnames=())
def kernel(positions, hidden_states, Wq, Wk, Wv, Wo, q_norm_w, k_norm_w,
           in_ln_w, post_ln_w, Wg, We_gate, We_up, We_down, Ws_gate, Ws_up,
           Ws_down):
    pos2d = positions.reshape(T, 1)

    ao = pl.pallas_call(
        _attn_kernel,
        grid=(NH, NRB),
        in_specs=[
            pl.BlockSpec((T, 1), lambda h, r: (0, 0)),
            pl.BlockSpec((T, H), lambda h, r: (0, 0)),
            pl.BlockSpec((H, NH * HD), lambda h, r: (0, 0)),
            pl.BlockSpec((H, NKV * HD), lambda h, r: (0, 0)),
            pl.BlockSpec((H, NKV * HD), lambda h, r: (0, 0)),
            pl.BlockSpec((HD,), lambda h, r: (0,)),
            pl.BlockSpec((HD,), lambda h, r: (0,)),
            pl.BlockSpec((H,), lambda h, r: (0,)),
        ],
        out_specs=pl.BlockSpec((1, RB, HD), lambda h, r: (h, r, 0)),
        out_shape=jax.ShapeDtypeStruct((NH, T, HD), jnp.bfloat16),
        scratch_shapes=[pltpu.VMEM((NH, T, HD), jnp.bfloat16),
                        pltpu.VMEM((NKV, T, HD), jnp.bfloat16),
                        pltpu.VMEM((NKV, T, HD), jnp.bfloat16)],
    )(pos2d, hidden_states, Wq, Wk, Wv, q_norm_w, k_norm_w, in_ln_w)

    out = pl.pallas_call(
        _moe_kernel,
        grid=(E,),
        in_specs=[
            pl.BlockSpec((NH, T, HD), lambda e: (0, 0, 0)),
            pl.BlockSpec((NH * HD, H), lambda e: (0, 0)),
            pl.BlockSpec((T, H), lambda e: (0, 0)),
            pl.BlockSpec((H,), lambda e: (0,)),
            pl.BlockSpec((H, E), lambda e: (0, 0)),
            pl.BlockSpec((1, H, DFF), lambda e: (e, 0, 0)),
            pl.BlockSpec((1, H, DFF), lambda e: (e, 0, 0)),
            pl.BlockSpec((1, DFF, H), lambda e: (e, 0, 0)),
            pl.BlockSpec((H, DFF), lambda e: (0, 0)),
            pl.BlockSpec((H, DFF), lambda e: (0, 0)),
            pl.BlockSpec((DFF, H), lambda e: (0, 0)),
        ],
        out_specs=pl.BlockSpec((T, H), lambda e: (0, 0)),
        out_shape=jax.ShapeDtypeStruct((T, H), jnp.float32),
        scratch_shapes=[pltpu.VMEM((T, H), jnp.bfloat16),
                        pltpu.VMEM((T, E), jnp.float32)],
    )(ao, Wo, hidden_states, post_ln_w, Wg,
      We_gate, We_up, We_down, Ws_gate, Ws_up, Ws_down)

    return out
